# baseline (speedup 1.0000x reference)
import jax
import jax.numpy as jnp
from jax.experimental import pallas as pl
from jax.experimental.pallas import tpu as pltpu

N = 4096
D_IN = 128
D_OUT = 128
BM = 512
CHUNKS = N // BM
NBUF = 3


def _spmm_kernel(x_hbm, w_hbm, adj_hbm, out_hbm,
                 xbuf, wbuf, xw, adjbuf, obuf,
                 sem_x, sem_w, sems_adj, sems_out):
    def adj_copy(chunk, slot):
        return pltpu.make_async_copy(
            adj_hbm.at[pl.ds(chunk * BM, BM), :], adjbuf.at[slot],
            sems_adj.at[slot])

    def out_copy(chunk, slot):
        return pltpu.make_async_copy(
            obuf.at[slot], out_hbm.at[pl.ds(chunk * BM, BM), :],
            sems_out.at[slot])

    adj_copy(0, 0).start()
    pltpu.make_async_copy(x_hbm, xbuf, sem_x).start()
    pltpu.make_async_copy(w_hbm, wbuf, sem_w).start()
    adj_copy(1, 1).start()

    pltpu.make_async_copy(x_hbm, xbuf, sem_x).wait()
    pltpu.make_async_copy(w_hbm, wbuf, sem_w).wait()
    xw[...] = jnp.dot(xbuf[...], wbuf[...],
                      preferred_element_type=jnp.float32)

    for chunk in range(CHUNKS):
        slot = chunk % NBUF
        adj_copy(chunk, slot).wait()
        if chunk + 2 < CHUNKS:
            adj_copy(chunk + 2, (chunk + 2) % NBUF).start()
        oslot = chunk % 2
        if chunk >= 2:
            out_copy(chunk - 2, oslot).wait()
        mask = (adjbuf[slot] != 0.0).astype(jnp.float32)
        obuf[oslot] = jnp.dot(mask, xw[...],
                              preferred_element_type=jnp.float32)
        out_copy(chunk, oslot).start()

    out_copy(CHUNKS - 2, (CHUNKS - 2) % 2).wait()
    out_copy(CHUNKS - 1, (CHUNKS - 1) % 2).wait()


def kernel(input, adj, weight):
    hbm = pl.BlockSpec(memory_space=pltpu.MemorySpace.HBM)
    return pl.pallas_call(
        _spmm_kernel,
        in_specs=[hbm, hbm, hbm],
        out_specs=hbm,
        out_shape=jax.ShapeDtypeStruct((N, D_OUT), jnp.float32),
        scratch_shapes=[
            pltpu.VMEM((N, D_IN), jnp.float32),
            pltpu.VMEM((D_IN, D_OUT), jnp.float32),
            pltpu.VMEM((N, D_OUT), jnp.float32),
            pltpu.VMEM((NBUF, BM, N), jnp.float32),
            pltpu.VMEM((2, BM, D_OUT), jnp.float32),
            pltpu.SemaphoreType.DMA,
            pltpu.SemaphoreType.DMA,
            pltpu.SemaphoreType.DMA((NBUF,)),
            pltpu.SemaphoreType.DMA((2,)),
        ],
    )(input, weight, adj)
